# Initial kernel scaffold; baseline (speedup 1.0000x reference)
import jax
import jax.numpy as jnp
from jax.experimental import pallas as pl


def kernel(root_cubes):
    raise NotImplementedError("write your pallas kernel here")


# speedup vs baseline: 11.0706x; 11.0706x over previous
import functools

import jax
import jax.numpy as jnp
from jax import lax
from jax.experimental import pallas as pl
from jax.experimental.pallas import tpu as pltpu

_X, _Y, _Z = 128, 128, 32
_C = _Y * _Z
_K = 10
_NEG = float("-inf")
_BIG = 2**30


def _nms_topk_kernel(x_ref, out_ref, nms_ref):
    a = x_ref[0]
    col = lax.broadcasted_iota(jnp.int32, (_X, _C), 1)
    z = jnp.bitwise_and(col, _Z - 1)
    neg_col = jnp.full((_X, 1), _NEG, jnp.float32)
    neg_y = jnp.full((_X, _Z), _NEG, jnp.float32)
    neg_row = jnp.full((1, _C), _NEG, jnp.float32)

    zp = jnp.concatenate([a[:, 1:], neg_col], axis=1)
    zp = jnp.where(z == _Z - 1, _NEG, zp)
    zm = jnp.concatenate([neg_col, a[:, :-1]], axis=1)
    zm = jnp.where(z == 0, _NEG, zm)
    mz = jnp.maximum(a, jnp.maximum(zp, zm))
    yp = jnp.concatenate([mz[:, _Z:], neg_y], axis=1)
    ym = jnp.concatenate([neg_y, mz[:, :-_Z]], axis=1)
    my = jnp.maximum(mz, jnp.maximum(yp, ym))
    xp = jnp.concatenate([my[1:], neg_row], axis=0)
    xm = jnp.concatenate([neg_row, my[:-1]], axis=0)
    m = jnp.maximum(my, jnp.maximum(xp, xm))

    nms = jnp.where(a == m, a, 0.0)
    nms_ref[...] = nms

    rowvals = jnp.max(nms, axis=1, keepdims=True)
    colidx = jnp.min(jnp.where(nms == rowvals, col, _BIG), axis=1, keepdims=True)
    rowio = lax.broadcasted_iota(jnp.int32, (_X, 1), 0)

    vals, rs, cs = [], [], []
    for _ in range(_K):
        gmax = jnp.max(rowvals)
        r = jnp.min(jnp.where(rowvals == gmax, rowio, _BIG))
        c = jnp.min(jnp.where(rowio == r, colidx, _BIG))
        vals.append(gmax)
        rs.append(r)
        cs.append(c)
        row = nms_ref[pl.ds(r, 1), :]
        lcol = lax.broadcasted_iota(jnp.int32, (1, _C), 1)
        rowm = jnp.where(lcol == c, _NEG, row)
        nms_ref[pl.ds(r, 1), :] = rowm
        nrv = jnp.max(rowm)
        nci = jnp.min(jnp.where(rowm == nrv, lcol, _BIG))
        rowvals = jnp.where(rowio == r, nrv, rowvals)
        colidx = jnp.where(rowio == r, nci, colidx)

    def row10(scalars, dtype=jnp.float32):
        return jnp.concatenate([s.astype(dtype).reshape(1, 1) for s in scalars], axis=1)

    fx = row10(rs)
    iy_l = [c // _Z for c in cs]
    iz_l = [c % _Z for c in cs]
    fy = row10(iy_l)
    fz = row10(iz_l)
    fv = row10(vals)
    locx = (fx / float(_X - 1) * 8000.0 + 0.0) - 4000.0
    locy = (fy / float(_Y - 1) * 8000.0 + 0.0) - 4000.0
    locz = (fz / float(_Z - 1) * 2000.0 + 800.0) - 1000.0
    flag = (fv > 0.3).astype(jnp.float32) - 1.0

    out5 = jnp.concatenate([locx, locy, locz, flag, fv], axis=0)
    out5 = jnp.concatenate([out5, jnp.zeros((5, 118), jnp.float32)], axis=1)
    out8 = jnp.concatenate([out5, jnp.zeros((3, 128), jnp.float32)], axis=0)
    out_ref[0] = out8


@jax.jit
def kernel(root_cubes):
    rc = lax.stop_gradient(root_cubes)
    b = rc.shape[0]
    a2 = rc.reshape(b, _X, _C)
    out = pl.pallas_call(
        _nms_topk_kernel,
        grid=(b,),
        in_specs=[pl.BlockSpec((1, _X, _C), lambda i: (i, 0, 0))],
        out_specs=pl.BlockSpec((1, 8, 128), lambda i: (i, 0, 0)),
        out_shape=jax.ShapeDtypeStruct((b, 8, 128), jnp.float32),
        scratch_shapes=[pltpu.VMEM((_X, _C), jnp.float32)],
    )(a2)
    return out[:, :5, :_K].transpose(0, 2, 1)


# speedup vs baseline: 15.1064x; 1.3645x over previous
import jax
import jax.numpy as jnp
from jax import lax
from jax.experimental import pallas as pl
from jax.experimental.pallas import tpu as pltpu
from jax.experimental.pallas import tpu_sc as plsc

_X, _Y, _Z = 128, 128, 32
_C = _Y * _Z
_K = 10
_NEG = float("-inf")
_BIG = 2**30


def _nms_kernel(x_ref, nms_ref, rv_ref, ci_ref):
    a = x_ref[0]
    col = lax.broadcasted_iota(jnp.int32, (_X, _C), 1)
    z = jnp.bitwise_and(col, _Z - 1)
    neg_col = jnp.full((_X, 1), _NEG, jnp.float32)
    neg_y = jnp.full((_X, _Z), _NEG, jnp.float32)
    neg_row = jnp.full((1, _C), _NEG, jnp.float32)

    zp = jnp.concatenate([a[:, 1:], neg_col], axis=1)
    zp = jnp.where(z == _Z - 1, _NEG, zp)
    zm = jnp.concatenate([neg_col, a[:, :-1]], axis=1)
    zm = jnp.where(z == 0, _NEG, zm)
    mz = jnp.maximum(a, jnp.maximum(zp, zm))
    yp = jnp.concatenate([mz[:, _Z:], neg_y], axis=1)
    ym = jnp.concatenate([neg_y, mz[:, :-_Z]], axis=1)
    my = jnp.maximum(mz, jnp.maximum(yp, ym))
    xp = jnp.concatenate([my[1:], neg_row], axis=0)
    xm = jnp.concatenate([neg_row, my[:-1]], axis=0)
    m = jnp.maximum(my, jnp.maximum(xp, xm))

    nms = jnp.where(a == m, a, 0.0)
    nms_ref[0] = nms

    rowvals = jnp.max(nms, axis=1, keepdims=True)
    colidx = jnp.min(jnp.where(nms == rowvals, col, _BIG), axis=1, keepdims=True)
    rv_ref[0] = rowvals
    ci_ref[0] = colidx


def _sc_topk(nms_hbm, rv_hbm, ci_hbm, out_hbm, rv_v, ci_v, row_v, out_v):
    wid = lax.axis_index("s") * 2 + lax.axis_index("c")
    lane = lax.iota(jnp.int32, 16)
    pltpu.sync_copy(rv_hbm.at[wid], rv_v)
    pltpu.sync_copy(ci_hbm.at[wid], ci_v)

    neg = jnp.full((16,), _NEG, jnp.float32)
    big = jnp.full((16,), _BIG, jnp.int32)

    _gdn = lax.GatherDimensionNumbers(
        offset_dims=(), collapsed_slice_dims=(0,), start_index_map=(0,)
    )

    def shuf(v, idx):
        return lax.gather(
            v, idx[:, None], _gdn, (1,),
            mode=lax.GatherScatterMode.PROMISE_IN_BOUNDS,
        )

    def bfly_max(v):
        for s in (1, 2, 4, 8):
            v = jnp.maximum(v, shuf(v, lane ^ s))
        return v

    def bfly_min_i(v):
        for s in (1, 2, 4, 8):
            v = jnp.minimum(v, shuf(v, lane ^ s))
        return v

    vals, rs, cs = [], [], []
    for _ in range(_K):
        bestv, besti = neg, big
        for k in range(_X // 16):
            v = rv_v[pl.ds(k * 16, 16)]
            idx = lane + (k * 16)
            take = (v > bestv) | ((v == bestv) & (idx < besti))
            bestv = jnp.where(take, v, bestv)
            besti = jnp.where(take, idx, besti)
        m = bfly_max(bestv)
        r = bfly_min_i(jnp.where(bestv == m, besti, _BIG))
        r_s = r[0]
        rb = (r_s // 16) * 16
        civ = ci_v[pl.ds(rb, 16)]
        c = bfly_min_i(jnp.where(lane + rb == r, civ, _BIG))

        dead_cols = [c] + [jnp.where(r == rj, cj, -1) for rj, cj in zip(rs, cs)]
        vals.append(m)
        rs.append(r)
        cs.append(c)

        pltpu.sync_copy(nms_hbm.at[wid, r_s], row_v)

        def body(k, carry):
            nbv, nbi = carry
            v = row_v[pl.ds(k * 16, 16)]
            cid = lane + k * 16
            dead = cid == dead_cols[0]
            for dc in dead_cols[1:]:
                dead = dead | (cid == dc)
            v = jnp.where(dead, _NEG, v)
            take = (v > nbv) | ((v == nbv) & (cid < nbi))
            return jnp.where(take, v, nbv), jnp.where(take, cid, nbi)

        nbv, nbi = lax.fori_loop(0, _C // 16, body, (neg, big))
        nrv = bfly_max(nbv)
        nci = bfly_min_i(jnp.where(nbv == nrv, nbi, _BIG))
        sel = lane + rb == r
        rv_v[pl.ds(rb, 16)] = jnp.where(sel, nrv, rv_v[pl.ds(rb, 16)])
        ci_v[pl.ds(rb, 16)] = jnp.where(sel, nci, ci_v[pl.ds(rb, 16)])

    def lanevec(splats, dtype):
        out = jnp.zeros((16,), dtype)
        for i, s in enumerate(splats):
            out = jnp.where(lane == i, s.astype(dtype), out)
        return out

    fv = lanevec(vals, jnp.float32)
    ix = lanevec(rs, jnp.int32)
    fc = lanevec(cs, jnp.int32)
    iy = lax.shift_right_logical(fc, 5)
    iz = jnp.bitwise_and(fc, _Z - 1)
    keep = lane < _K
    locx = (ix.astype(jnp.float32) / float(_X - 1) * 8000.0 + 0.0) - 4000.0
    locy = (iy.astype(jnp.float32) / float(_Y - 1) * 8000.0 + 0.0) - 4000.0
    locz = (iz.astype(jnp.float32) / float(_Z - 1) * 2000.0 + 800.0) - 1000.0
    flag = jnp.where(fv > 0.3, 0.0, -1.0)
    for f, vec in enumerate([locx, locy, locz, flag, fv]):
        out_v[pl.ds(f * 16, 16)] = jnp.where(keep, vec, 0.0)
    pltpu.sync_copy(out_v, out_hbm.at[wid])


@jax.jit
def kernel(root_cubes):
    rc = lax.stop_gradient(root_cubes)
    b = rc.shape[0]
    a2 = rc.reshape(b, _X, _C)
    nms, rv, ci = pl.pallas_call(
        _nms_kernel,
        grid=(b,),
        in_specs=[pl.BlockSpec((1, _X, _C), lambda i: (i, 0, 0))],
        out_specs=[
            pl.BlockSpec((1, _X, _C), lambda i: (i, 0, 0)),
            pl.BlockSpec((1, _X, 1), lambda i: (i, 0, 0)),
            pl.BlockSpec((1, _X, 1), lambda i: (i, 0, 0)),
        ],
        out_shape=[
            jax.ShapeDtypeStruct((b, _X, _C), jnp.float32),
            jax.ShapeDtypeStruct((b, _X, 1), jnp.float32),
            jax.ShapeDtypeStruct((b, _X, 1), jnp.int32),
        ],
    )(a2)

    mesh = plsc.VectorSubcoreMesh(core_axis_name="c", subcore_axis_name="s")
    out = pl.kernel(
        _sc_topk,
        mesh=mesh,
        out_type=jax.ShapeDtypeStruct((b, 80), jnp.float32),
        scratch_types=[
            pltpu.VMEM((_X,), jnp.float32),
            pltpu.VMEM((_X,), jnp.int32),
            pltpu.VMEM((_C,), jnp.float32),
            pltpu.VMEM((80,), jnp.float32),
        ],
    )(nms, rv.reshape(b, _X), ci.reshape(b, _X))
    return out.reshape(b, 5, 16)[:, :, :_K].transpose(0, 2, 1)
